# baseline (speedup 1.0000x reference)
import functools

import jax
import jax.numpy as jnp
from jax import lax
from jax.experimental import pallas as pl
from jax.experimental.pallas import tpu as pltpu
from jax.experimental.pallas import tpu_sc as plsc

_N = 100000
_H = 64
_TXN_IN = 256

_TB = 512


def _pairize_body(x_ref, o_ref):
    t = jnp.transpose(x_ref[...])
    o_ref[...] = jnp.concatenate([t[0:_TB], t[_TB:2 * _TB]], axis=1)


def _pairize(tT):
    nblk = (tT.shape[1] + 2 * _TB - 1) // (2 * _TB)
    return pl.pallas_call(
        _pairize_body,
        grid=(nblk,),
        in_specs=[pl.BlockSpec((_H, 2 * _TB), lambda i: (0, i))],
        out_specs=pl.BlockSpec((_TB, 128), lambda i: (i, 0)),
        out_shape=jax.ShapeDtypeStruct((nblk * _TB, 128), jnp.float32),
        compiler_params=pltpu.CompilerParams(
            dimension_semantics=("arbitrary",),
        ),
    )(tT)


_SLOT = 1024
_HALF = 512
_SUB = 128
_NPAD = 100352
_NSLOTS = _NPAD // _SLOT
_NW = 32

_sc_mesh = plsc.VectorSubcoreMesh(core_axis_name="c", subcore_axis_name="s")


@functools.partial(
    pl.kernel,
    mesh=_sc_mesh,
    out_type=[jax.ShapeDtypeStruct((_NPAD, 128), jnp.float32)] * 3,
    scratch_types=[
        pltpu.VMEM((8, _SUB), jnp.int32),
        pltpu.VMEM((_HALF, 128), jnp.float32),
        pltpu.SemaphoreType.DMA,
    ],
)
def _gather3(idx_c, idx_a, idx_e, mem_c, mem_a, mem_e,
             out_c, out_a, out_e, idx_v, rows_v, sem):
    wid = lax.axis_index("s") * 2 + lax.axis_index("c")
    for idx_hbm, mem_hbm, out_hbm in ((idx_c, mem_c, out_c),
                                      (idx_a, mem_a, out_a),
                                      (idx_e, mem_e, out_e)):
        for j in range((_NSLOTS + _NW - 1) // _NW):
            c = wid + _NW * j

            @pl.when(c < _NSLOTS)
            def _():
                pltpu.sync_copy(idx_hbm.at[pl.ds(c * 8, 8)], idx_v)
                for h in range(2):
                    cps = [
                        pltpu.async_copy(
                            mem_hbm.at[idx_v.at[4 * h + q]],
                            rows_v.at[pl.ds(q * _SUB, _SUB)],
                            sem,
                        )
                        for q in range(4)
                    ]
                    for cp in cps:
                        cp.wait()
                    pltpu.sync_copy(
                        rows_v,
                        out_hbm.at[pl.ds(c * _SLOT + h * _HALF, _HALF)])


_BR = 1000


def _txn_body(x_ref, w1_ref, b1_ref, wv1a_ref, bv1_ref, a_ref):
    x = x_ref[...].astype(jnp.bfloat16)
    h = jnp.maximum(
        jnp.dot(x, w1_ref[...], preferred_element_type=jnp.float32) + b1_ref[...],
        0.0).astype(jnp.bfloat16)
    a_ref[...] = (
        jnp.dot(h, wv1a_ref[...], preferred_element_type=jnp.float32)
        + bv1_ref[...])


def _txn_part(txn_x, W1, b1, Wv1a, bv1):
    grid = _N // _BR
    return pl.pallas_call(
        _txn_body,
        grid=(grid,),
        in_specs=[
            pl.BlockSpec((_BR, _TXN_IN), lambda i: (i, 0)),
            pl.BlockSpec((_TXN_IN, _H), lambda i: (0, 0)),
            pl.BlockSpec((1, _H), lambda i: (0, 0)),
            pl.BlockSpec((_H, _H), lambda i: (0, 0)),
            pl.BlockSpec((1, _H), lambda i: (0, 0)),
        ],
        out_specs=pl.BlockSpec((_BR, _H), lambda i: (i, 0)),
        out_shape=jax.ShapeDtypeStruct((_N, _H), jnp.float32),
        compiler_params=pltpu.CompilerParams(
            dimension_semantics=("arbitrary",),
        ),
    )(txn_x, W1, b1, Wv1a, bv1)


def _pick_row(g2, p2):
    return jnp.where(p2 == 0, g2[:, :_H], g2[:, _H:]).astype(jnp.bfloat16)


def _tail_body(a_ref, gc_ref, ga_ref, ge_ref, pc_ref, pa_ref, pe_ref,
               wv1g_ref, wv2_ref, bv2_ref, out_ref):
    acc = a_ref[...]
    for k, (g_ref, p_ref) in enumerate(
            ((gc_ref, pc_ref), (ga_ref, pa_ref), (ge_ref, pe_ref))):
        g = _pick_row(g_ref[...], p_ref[0])
        acc += jnp.dot(g, wv1g_ref[k * _H:(k + 1) * _H, :],
                       preferred_element_type=jnp.float32)
    z = jnp.maximum(acc, 0.0)
    out_ref[...] = (
        jnp.dot(z, wv2_ref[...], preferred_element_type=jnp.float32)
        + bv2_ref[...])


def _tail(a, gc, ga, ge, pc, pa, pe, Wv1g, Wv2, bv2):
    grid = _N // _BR
    g_spec = pl.BlockSpec((_BR, 128), lambda i: (i, 0))
    p_spec = pl.BlockSpec((1, _BR, 1), lambda i: (i, 0, 0))
    return pl.pallas_call(
        _tail_body,
        grid=(grid,),
        in_specs=[
            pl.BlockSpec((_BR, _H), lambda i: (i, 0)),
            g_spec, g_spec, g_spec,
            p_spec, p_spec, p_spec,
            pl.BlockSpec((3 * _H, _H), lambda i: (0, 0)),
            pl.BlockSpec((_H, 1), lambda i: (0, 0)),
            pl.BlockSpec((1, 1), lambda i: (0, 0)),
        ],
        out_specs=pl.BlockSpec((_BR, 1), lambda i: (i, 0)),
        out_shape=jax.ShapeDtypeStruct((_N, 1), jnp.float32),
        compiler_params=pltpu.CompilerParams(
            dimension_semantics=("arbitrary",),
        ),
    )(a, gc, ga, ge, pc, pa, pe, Wv1g, Wv2, bv2)


def kernel(txn_x, idx_card, idx_addr, idx_email, mem_card, mem_addr, mem_email,
           W1, b1, unk_card, unk_addr, unk_email, Wv1, bv1, Wv2, bv2):
    pad = _NPAD - _N
    idx4 = []
    pmod = []
    for i in (idx_card, idx_addr, idx_email):
        i = i.astype(jnp.int32)
        w = i % (2 * _TB)
        p = (i // (2 * _TB)) * _TB + (w % _TB)
        idx4.append(jnp.pad(p, (0, pad)).reshape(_NPAD // _SUB, _SUB))
        pmod.append((w // _TB).reshape(_N // _BR, _BR, 1))
    m4 = [_pairize(m.T) for m in (mem_card, mem_addr, mem_email)]
    wv1b = Wv1.astype(jnp.bfloat16)
    gc, ga, ge = _gather3(idx4[0], idx4[1], idx4[2], m4[0], m4[1], m4[2])
    a = _txn_part(txn_x, W1.astype(jnp.bfloat16), b1.reshape(1, _H),
                  wv1b[0:_H], bv1.reshape(1, _H))
    out = _tail(a, gc, ga, ge, pmod[0], pmod[1], pmod[2],
                wv1b[_H:], Wv2, bv2.reshape(1, 1))
    return out[:, 0]


# speedup vs baseline: 1.2570x; 1.2570x over previous
import functools

import jax
import jax.numpy as jnp
from jax import lax
from jax.experimental import pallas as pl
from jax.experimental.pallas import tpu as pltpu
from jax.experimental.pallas import tpu_sc as plsc

_N = 100000
_H = 64
_TXN_IN = 256

_CHUNK = 1024
_SUB = 128
_NPAD = 100352
_NCHUNKS = _NPAD // _CHUNK
_NW = 32

_sc_mesh = plsc.VectorSubcoreMesh(core_axis_name="c", subcore_axis_name="s")


def _gather_tables(idx_refs, mem_refs, out_refs, idx_v, rows_v, sem, wid):
    for idx_hbm, mem_hbm, out_hbm in zip(idx_refs, mem_refs, out_refs):
        for j in range((_NCHUNKS + _NW - 1) // _NW):
            c = wid + _NW * j

            @pl.when(c < _NCHUNKS)
            def _():
                pltpu.sync_copy(idx_hbm.at[pl.ds(c * 8, 8)], idx_v)
                cps = [
                    pltpu.async_copy(
                        mem_hbm.at[idx_v.at[q]],
                        rows_v.at[pl.ds(q * _SUB, _SUB)],
                        sem,
                    )
                    for q in range(8)
                ]
                for cp in cps:
                    cp.wait()
                pltpu.sync_copy(rows_v, out_hbm.at[pl.ds(c * _CHUNK, _CHUNK)])


_sc_scratch = [
    pltpu.VMEM((8, _SUB), jnp.int32),
    pltpu.VMEM((_CHUNK, _H), jnp.float32),
    pltpu.SemaphoreType.DMA,
]
_sc_params = pltpu.CompilerParams(use_tc_tiling_on_sc=False)


@functools.partial(
    pl.kernel, mesh=_sc_mesh,
    out_type=[jax.ShapeDtypeStruct((_NPAD, _H), jnp.float32)] * 2,
    scratch_types=_sc_scratch, compiler_params=_sc_params,
)
def _gather_small(idx_a, idx_e, mem_a, mem_e, out_a, out_e,
                  idx_v, rows_v, sem):
    wid = lax.axis_index("s") * 2 + lax.axis_index("c")
    _gather_tables((idx_a, idx_e), (mem_a, mem_e), (out_a, out_e),
                   idx_v, rows_v, sem, wid)


@functools.partial(
    pl.kernel, mesh=_sc_mesh,
    out_type=[jax.ShapeDtypeStruct((_NPAD, _H), jnp.float32)],
    scratch_types=_sc_scratch, compiler_params=_sc_params,
)
def _gather_card(idx_c, mem_c, out_c, idx_v, rows_v, sem):
    wid = lax.axis_index("s") * 2 + lax.axis_index("c")
    _gather_tables((idx_c,), (mem_c,), (out_c,), idx_v, rows_v, sem, wid)


_BR = 1000


def _txn_body(x_ref, w1_ref, b1_ref, wv1a_ref, bv1_ref, a_ref):
    x = x_ref[...].astype(jnp.bfloat16)
    h = jnp.maximum(
        jnp.dot(x, w1_ref[...], preferred_element_type=jnp.float32) + b1_ref[...],
        0.0).astype(jnp.bfloat16)
    a_ref[...] = (
        jnp.dot(h, wv1a_ref[...], preferred_element_type=jnp.float32)
        + bv1_ref[...])


def _txn_part(txn_x, W1, b1, Wv1a, bv1):
    grid = _N // _BR
    return pl.pallas_call(
        _txn_body,
        grid=(grid,),
        in_specs=[
            pl.BlockSpec((_BR, _TXN_IN), lambda i: (i, 0)),
            pl.BlockSpec((_TXN_IN, _H), lambda i: (0, 0)),
            pl.BlockSpec((1, _H), lambda i: (0, 0)),
            pl.BlockSpec((_H, _H), lambda i: (0, 0)),
            pl.BlockSpec((1, _H), lambda i: (0, 0)),
        ],
        out_specs=pl.BlockSpec((_BR, _H), lambda i: (i, 0)),
        out_shape=jax.ShapeDtypeStruct((_N, _H), jnp.float32),
        compiler_params=pltpu.CompilerParams(
            dimension_semantics=("arbitrary",),
        ),
    )(txn_x, W1, b1, Wv1a, bv1)


def _tail_body(a_ref, gc_ref, ga_ref, ge_ref,
               wv1g_ref, wv2_ref, bv2_ref, out_ref):
    acc = a_ref[...]
    for k, g_ref in enumerate((gc_ref, ga_ref, ge_ref)):
        acc += jnp.dot(g_ref[...].astype(jnp.bfloat16),
                       wv1g_ref[k * _H:(k + 1) * _H, :],
                       preferred_element_type=jnp.float32)
    z = jnp.maximum(acc, 0.0)
    out_ref[...] = (
        jnp.dot(z, wv2_ref[...], preferred_element_type=jnp.float32)
        + bv2_ref[...])


def _tail(a, gc, ga, ge, Wv1g, Wv2, bv2):
    grid = _N // _BR
    g_spec = pl.BlockSpec((_BR, _H), lambda i: (i, 0))
    return pl.pallas_call(
        _tail_body,
        grid=(grid,),
        in_specs=[
            pl.BlockSpec((_BR, _H), lambda i: (i, 0)),
            g_spec, g_spec, g_spec,
            pl.BlockSpec((3 * _H, _H), lambda i: (0, 0)),
            pl.BlockSpec((_H, 1), lambda i: (0, 0)),
            pl.BlockSpec((1, 1), lambda i: (0, 0)),
        ],
        out_specs=pl.BlockSpec((_BR, 1), lambda i: (i, 0)),
        out_shape=jax.ShapeDtypeStruct((_N, 1), jnp.float32),
        compiler_params=pltpu.CompilerParams(
            dimension_semantics=("arbitrary",),
        ),
    )(a, gc, ga, ge, Wv1g, Wv2, bv2)


def kernel(txn_x, idx_card, idx_addr, idx_email, mem_card, mem_addr, mem_email,
           W1, b1, unk_card, unk_addr, unk_email, Wv1, bv1, Wv2, bv2):
    pad = _NPAD - _N
    idx2d = [
        jnp.pad(i.astype(jnp.int32), (0, pad)).reshape(_NPAD // _SUB, _SUB)
        for i in (idx_card, idx_addr, idx_email)
    ]
    wv1b = Wv1.astype(jnp.bfloat16)
    ga, ge = _gather_small(idx2d[1], idx2d[2], mem_addr, mem_email)
    (gc,) = _gather_card(idx2d[0], mem_card)
    a = _txn_part(txn_x, W1.astype(jnp.bfloat16), b1.reshape(1, _H),
                  wv1b[0:_H], bv1.reshape(1, _H))
    out = _tail(a, gc, ga, ge, wv1b[_H:], Wv2, bv2.reshape(1, 1))
    return out[:, 0]


# speedup vs baseline: 1.2798x; 1.0181x over previous
import functools

import jax
import jax.numpy as jnp
from jax import lax
from jax.experimental import pallas as pl
from jax.experimental.pallas import tpu as pltpu
from jax.experimental.pallas import tpu_sc as plsc

_N = 100000
_H = 64
_TXN_IN = 256

_CHUNK = 1024
_SUB = 128
_NCHUNKS = (_N + _CHUNK - 1) // _CHUNK
_LASTBASE = _N - _CHUNK
_NW = 32

_sc_mesh = plsc.VectorSubcoreMesh(core_axis_name="c", subcore_axis_name="s")


def _gather_tables(idx_refs, mem_refs, out_refs, idx_v, rows_v, sem, wid):
    for idx_hbm, mem_hbm, out_hbm in zip(idx_refs, mem_refs, out_refs):
        for j in range((_NCHUNKS + _NW - 1) // _NW):
            c = wid + _NW * j

            @pl.when(c < _NCHUNKS)
            def _():
                base = jnp.minimum(c * _CHUNK, _LASTBASE)
                pltpu.sync_copy(idx_hbm.at[pl.ds(base, _CHUNK)], idx_v)
                cps = [
                    pltpu.async_copy(
                        mem_hbm.at[idx_v.at[pl.ds(q * _SUB, _SUB)]],
                        rows_v.at[pl.ds(q * _SUB, _SUB)],
                        sem,
                    )
                    for q in range(8)
                ]
                for cp in cps:
                    cp.wait()
                pltpu.sync_copy(rows_v, out_hbm.at[pl.ds(base, _CHUNK)])


_sc_scratch = [
    pltpu.VMEM((_CHUNK,), jnp.int32),
    pltpu.VMEM((_CHUNK, _H), jnp.float32),
    pltpu.SemaphoreType.DMA,
]
_sc_params = pltpu.CompilerParams(use_tc_tiling_on_sc=False)


@functools.partial(
    pl.kernel, mesh=_sc_mesh,
    out_type=[jax.ShapeDtypeStruct((_N, _H), jnp.float32)] * 2,
    scratch_types=_sc_scratch, compiler_params=_sc_params,
)
def _gather_small(idx_a, idx_e, mem_a, mem_e, out_a, out_e,
                  idx_v, rows_v, sem):
    wid = lax.axis_index("s") * 2 + lax.axis_index("c")
    _gather_tables((idx_a, idx_e), (mem_a, mem_e), (out_a, out_e),
                   idx_v, rows_v, sem, wid)


@functools.partial(
    pl.kernel, mesh=_sc_mesh,
    out_type=[jax.ShapeDtypeStruct((_N, _H), jnp.float32)],
    scratch_types=_sc_scratch, compiler_params=_sc_params,
)
def _gather_card(idx_c, mem_c, out_c, idx_v, rows_v, sem):
    wid = lax.axis_index("s") * 2 + lax.axis_index("c")
    _gather_tables((idx_c,), (mem_c,), (out_c,), idx_v, rows_v, sem, wid)


_BR = 1000


def _txn_body(x_ref, w1_ref, b1_ref, wv1a_ref, bv1_ref, a_ref):
    x = x_ref[...].astype(jnp.bfloat16)
    h = jnp.maximum(
        jnp.dot(x, w1_ref[...], preferred_element_type=jnp.float32) + b1_ref[...],
        0.0).astype(jnp.bfloat16)
    a_ref[...] = (
        jnp.dot(h, wv1a_ref[...], preferred_element_type=jnp.float32)
        + bv1_ref[...])


def _txn_part(txn_x, W1, b1, Wv1a, bv1):
    grid = _N // _BR
    return pl.pallas_call(
        _txn_body,
        grid=(grid,),
        in_specs=[
            pl.BlockSpec((_BR, _TXN_IN), lambda i: (i, 0)),
            pl.BlockSpec((_TXN_IN, _H), lambda i: (0, 0)),
            pl.BlockSpec((1, _H), lambda i: (0, 0)),
            pl.BlockSpec((_H, _H), lambda i: (0, 0)),
            pl.BlockSpec((1, _H), lambda i: (0, 0)),
        ],
        out_specs=pl.BlockSpec((_BR, _H), lambda i: (i, 0)),
        out_shape=jax.ShapeDtypeStruct((_N, _H), jnp.float32),
        compiler_params=pltpu.CompilerParams(
            dimension_semantics=("arbitrary",),
        ),
    )(txn_x, W1, b1, Wv1a, bv1)


def _tail_body(a_ref, gc_ref, ga_ref, ge_ref,
               wv1g_ref, wv2_ref, bv2_ref, out_ref):
    acc = a_ref[...]
    for k, g_ref in enumerate((gc_ref, ga_ref, ge_ref)):
        acc += jnp.dot(g_ref[...].astype(jnp.bfloat16),
                       wv1g_ref[k * _H:(k + 1) * _H, :],
                       preferred_element_type=jnp.float32)
    z = jnp.maximum(acc, 0.0)
    out_ref[...] = (
        jnp.dot(z, wv2_ref[...], preferred_element_type=jnp.float32)
        + bv2_ref[...])


def _tail(a, gc, ga, ge, Wv1g, Wv2, bv2):
    grid = _N // _BR
    g_spec = pl.BlockSpec((_BR, _H), lambda i: (i, 0))
    return pl.pallas_call(
        _tail_body,
        grid=(grid,),
        in_specs=[
            pl.BlockSpec((_BR, _H), lambda i: (i, 0)),
            g_spec, g_spec, g_spec,
            pl.BlockSpec((3 * _H, _H), lambda i: (0, 0)),
            pl.BlockSpec((_H, 1), lambda i: (0, 0)),
            pl.BlockSpec((1, 1), lambda i: (0, 0)),
        ],
        out_specs=pl.BlockSpec((_BR, 1), lambda i: (i, 0)),
        out_shape=jax.ShapeDtypeStruct((_N, 1), jnp.float32),
        compiler_params=pltpu.CompilerParams(
            dimension_semantics=("arbitrary",),
        ),
    )(a, gc, ga, ge, Wv1g, Wv2, bv2)


def kernel(txn_x, idx_card, idx_addr, idx_email, mem_card, mem_addr, mem_email,
           W1, b1, unk_card, unk_addr, unk_email, Wv1, bv1, Wv2, bv2):
    wv1b = Wv1.astype(jnp.bfloat16)
    a = _txn_part(txn_x, W1.astype(jnp.bfloat16), b1.reshape(1, _H),
                  wv1b[0:_H], bv1.reshape(1, _H))
    ic, ia, ie = (i.astype(jnp.int32)
                  for i in (idx_card, idx_addr, idx_email))
    ga, ge = _gather_small(ia, ie, mem_addr, mem_email)
    (gc,) = _gather_card(ic, mem_card)
    out = _tail(a, gc, ga, ge, wv1b[_H:], Wv2, bv2.reshape(1, 1))
    return out[:, 0]


# speedup vs baseline: 1.3038x; 1.0188x over previous
import functools

import jax
import jax.numpy as jnp
from jax import lax
from jax.experimental import pallas as pl
from jax.experimental.pallas import tpu as pltpu
from jax.experimental.pallas import tpu_sc as plsc

_N = 100000
_H = 64
_TXN_IN = 256

_CHUNK = 1024
_SUB = 128
_NCHUNKS = (_N + _CHUNK - 1) // _CHUNK
_LASTBASE = _N - _CHUNK
_NW = 32

_sc_mesh = plsc.VectorSubcoreMesh(core_axis_name="c", subcore_axis_name="s")


def _gather_tables(idx_refs, mem_refs, out_refs, idx_v, rows_v, sem, wid):
    for idx_hbm, mem_hbm, out_hbm in zip(idx_refs, mem_refs, out_refs):
        for j in range((_NCHUNKS + _NW - 1) // _NW):
            c = wid + _NW * j

            @pl.when(c < _NCHUNKS)
            def _():
                base = jnp.minimum(c * _CHUNK, _LASTBASE)
                pltpu.sync_copy(idx_hbm.at[pl.ds(base, _CHUNK)], idx_v)
                cps = [
                    pltpu.async_copy(
                        mem_hbm.at[idx_v.at[pl.ds(q * _SUB, _SUB)]],
                        rows_v.at[pl.ds(q * _SUB, _SUB)],
                        sem,
                    )
                    for q in range(8)
                ]
                for cp in cps:
                    cp.wait()
                pltpu.sync_copy(rows_v, out_hbm.at[pl.ds(base, _CHUNK)])


_sc_scratch = [
    pltpu.VMEM((_CHUNK,), jnp.int32),
    pltpu.VMEM((_CHUNK, _H), jnp.float32),
    pltpu.SemaphoreType.DMA,
]
_sc_params = pltpu.CompilerParams(use_tc_tiling_on_sc=False)


@functools.partial(
    pl.kernel, mesh=_sc_mesh,
    out_type=[jax.ShapeDtypeStruct((_N, _H), jnp.float32)] * 2,
    scratch_types=_sc_scratch, compiler_params=_sc_params,
)
def _gather_small(idx_a, idx_e, mem_a, mem_e, out_a, out_e,
                  idx_v, rows_v, sem):
    wid = lax.axis_index("s") * 2 + lax.axis_index("c")
    _gather_tables((idx_a, idx_e), (mem_a, mem_e), (out_a, out_e),
                   idx_v, rows_v, sem, wid)


@functools.partial(
    pl.kernel, mesh=_sc_mesh,
    out_type=[jax.ShapeDtypeStruct((_N, _H), jnp.float32)],
    scratch_types=_sc_scratch, compiler_params=_sc_params,
)
def _gather_card(idx_c, mem_c, out_c, idx_v, rows_v, sem):
    wid = lax.axis_index("s") * 2 + lax.axis_index("c")
    _gather_tables((idx_c,), (mem_c,), (out_c,), idx_v, rows_v, sem, wid)


_BR = 1000


_PBR = 400


def _txn_body(x_ref, w1_ref, b1_ref, wv1a_ref, bv1_ref, a_ref):
    x = x_ref[...].astype(jnp.bfloat16)
    h = jnp.maximum(
        jnp.dot(x, w1_ref[...], preferred_element_type=jnp.float32) + b1_ref[...],
        0.0).astype(jnp.bfloat16)
    a_ref[...] = (
        jnp.dot(h, wv1a_ref[...], preferred_element_type=jnp.float32)
        + bv1_ref[...])


def _txn_part(txn_x2, W1bd, b1c, Wv1abd, bv1c):
    grid = _N // 2 // _PBR
    return pl.pallas_call(
        _txn_body,
        grid=(grid,),
        in_specs=[
            pl.BlockSpec((_PBR, 2 * _TXN_IN), lambda i: (i, 0)),
            pl.BlockSpec((2 * _TXN_IN, 128), lambda i: (0, 0)),
            pl.BlockSpec((1, 128), lambda i: (0, 0)),
            pl.BlockSpec((128, 128), lambda i: (0, 0)),
            pl.BlockSpec((1, 128), lambda i: (0, 0)),
        ],
        out_specs=pl.BlockSpec((_PBR, 128), lambda i: (i, 0)),
        out_shape=jax.ShapeDtypeStruct((_N // 2, 128), jnp.float32),
        compiler_params=pltpu.CompilerParams(
            dimension_semantics=("arbitrary",),
        ),
    )(txn_x2, W1bd, b1c, Wv1abd, bv1c)


def _tail_body(a_ref, gc_ref, ga_ref, ge_ref,
               wv1g_ref, wv2_ref, bv2_ref, out_ref):
    acc = a_ref[...]
    for k, g_ref in enumerate((gc_ref, ga_ref, ge_ref)):
        g = jnp.reshape(g_ref[...], (_PBR, 128)).astype(jnp.bfloat16)
        acc += jnp.dot(g, wv1g_ref[k * 128:(k + 1) * 128, :],
                       preferred_element_type=jnp.float32)
    z = jnp.maximum(acc, 0.0)
    out_ref[...] = (
        jnp.dot(z, wv2_ref[...], preferred_element_type=jnp.float32)
        + bv2_ref[...])


def _tail(a2, gc, ga, ge, Wv1gbd, Wv2bd, bv2):
    grid = _N // 2 // _PBR
    g_spec = pl.BlockSpec((_PBR * 128,), lambda i: (i,))
    return pl.pallas_call(
        _tail_body,
        grid=(grid,),
        in_specs=[
            pl.BlockSpec((_PBR, 128), lambda i: (i, 0)),
            g_spec, g_spec, g_spec,
            pl.BlockSpec((3 * 128, 128), lambda i: (0, 0)),
            pl.BlockSpec((128, 2), lambda i: (0, 0)),
            pl.BlockSpec((1, 1), lambda i: (0, 0)),
        ],
        out_specs=pl.BlockSpec((_PBR, 2), lambda i: (i, 0)),
        out_shape=jax.ShapeDtypeStruct((_N // 2, 2), jnp.float32),
        compiler_params=pltpu.CompilerParams(
            dimension_semantics=("arbitrary",),
        ),
    )(a2, gc, ga, ge, Wv1gbd, Wv2bd, bv2)


def _blockdiag(w):
    z = jnp.zeros_like(w)
    return jnp.concatenate(
        [jnp.concatenate([w, z], axis=1), jnp.concatenate([z, w], axis=1)],
        axis=0)


def kernel(txn_x, idx_card, idx_addr, idx_email, mem_card, mem_addr, mem_email,
           W1, b1, unk_card, unk_addr, unk_email, Wv1, bv1, Wv2, bv2):
    wv1b = Wv1.astype(jnp.bfloat16)
    w1b = W1.astype(jnp.bfloat16)
    b1c = jnp.concatenate([b1, b1]).reshape(1, 128)
    bv1c = jnp.concatenate([bv1, bv1]).reshape(1, 128)
    wv1gbd = jnp.concatenate(
        [_blockdiag(wv1b[(k + 1) * _H:(k + 2) * _H]) for k in range(3)],
        axis=0)
    wv2bd = jnp.concatenate(
        [jnp.concatenate([Wv2, jnp.zeros_like(Wv2)], axis=1),
         jnp.concatenate([jnp.zeros_like(Wv2), Wv2], axis=1)], axis=0)
    a2 = _txn_part(txn_x.reshape(_N // 2, 2 * _TXN_IN),
                   _blockdiag(w1b), b1c, _blockdiag(wv1b[0:_H]), bv1c)
    ic, ia, ie = (i.astype(jnp.int32)
                  for i in (idx_card, idx_addr, idx_email))
    ga, ge = _gather_small(ia, ie, mem_addr, mem_email)
    (gc,) = _gather_card(ic, mem_card)
    out = _tail(a2, gc.reshape(-1), ga.reshape(-1), ge.reshape(-1),
                wv1gbd, wv2bd, bv2.reshape(1, 1))
    return out.reshape(_N)
